# baseline (speedup 1.0000x reference)
import functools

import jax
import jax.numpy as jnp
from jax import lax
from jax.experimental import pallas as pl
from jax.experimental.pallas import tpu as pltpu
from jax.experimental.pallas import tpu_sc as plsc

N_NODES = 10000
NPAD = 10240
IN_CH = 128
HIDDEN = 128
HEADS = 2
TEMP = 0.7

EP1 = 331776
EP1_PER_TILE = EP1 // 16
EP1_CHUNKS = EP1_PER_TILE // 128

EP2 = 327680
EP2_PER_W = EP2 // 32
EP2_GROUPS = 20

NB = 512
NBLK = NPAD // NB


def _tc_a_body(x_ref, w_ref, xt_ref, pt_ref, xp0_ref, xp1_ref, av_ref):
    r = jnp.dot(x_ref[...], w_ref[...], preferred_element_type=jnp.float32)
    xp0_ref[...] = r[:, :HIDDEN]
    xp1_ref[...] = r[:, HIDDEN:]
    av_ref[...] = jnp.dot(pt_ref[...], xt_ref[...],
                          preferred_element_type=jnp.float32)


def _tc_a(x_pad, W, xT, pT):
    return pl.pallas_call(
        _tc_a_body,
        grid=(NBLK,),
        in_specs=[
            pl.BlockSpec((NB, IN_CH), lambda i: (i, 0)),
            pl.BlockSpec((IN_CH, HEADS * HIDDEN), lambda i: (0, 0)),
            pl.BlockSpec((IN_CH, NB), lambda i: (0, i)),
            pl.BlockSpec((4, IN_CH), lambda i: (0, 0)),
        ],
        out_specs=[
            pl.BlockSpec((NB, HIDDEN), lambda i: (i, 0)),
            pl.BlockSpec((NB, HIDDEN), lambda i: (i, 0)),
            pl.BlockSpec((4, NB), lambda i: (0, i)),
        ],
        out_shape=[
            jax.ShapeDtypeStruct((NPAD, HIDDEN), jnp.float32),
            jax.ShapeDtypeStruct((NPAD, HIDDEN), jnp.float32),
            jax.ShapeDtypeStruct((4, NPAD), jnp.float32),
        ],
    )(x_pad, W, xT, pT)


IBLK = 2304
NIB = EP1_PER_TILE // IBLK
NCH = IBLK // 128


def _sc_b_body(av, xp0, xp1, srch, dsth, out0, out1, d0, d1,
               as_v, ad_v, si2, di2, srcb, dstb, sbuf, xbuf, zrow,
               den_sh, acc_sh, gsem):
    c = lax.axis_index("c")
    s = lax.axis_index("s")
    zero16 = jnp.zeros((16,), jnp.float32)

    @pl.when(c == 0)
    def _():
        pltpu.sync_copy(av.at[0], as_v)
        pltpu.sync_copy(av.at[2], ad_v)

    @pl.when(c == 1)
    def _():
        pltpu.sync_copy(av.at[1], as_v)
        pltpu.sync_copy(av.at[3], ad_v)

    for i in range(40):
        zrow[pl.ds(16 * i, 16)] = zero16

    def _zrow_body(r, _):
        for q in range(8):
            xbuf[r, pl.ds(16 * q, 16)] = zero16
        return 0

    lax.fori_loop(0, 128, _zrow_body, 0)

    pltpu.sync_copy(zrow, den_sh.at[pl.ds(s * 640, 640)])
    for jj in range(5):
        pltpu.sync_copy(xbuf, acc_sh.at[pl.ds(s * 640 + jj * 128, 128)])
    plsc.subcore_barrier()

    ebase = s * EP1_PER_TILE
    iota16 = lax.iota(jnp.int32, 16)

    def _iblk(jo, _):
        pltpu.sync_copy(srch.at[pl.ds(ebase + jo * IBLK, IBLK)], si2)
        pltpu.sync_copy(dsth.at[pl.ds(ebase + jo * IBLK, IBLK)], di2)

        def _chunk(j, _):
            off = j * 128
            for k in range(8):
                srcb[pl.ds(16 * k, 16)] = si2[pl.ds(off + 16 * k, 16)]
                dstb[pl.ds(16 * k, 16)] = di2[pl.ds(off + 16 * k, 16)]

            @pl.when(c == 0)
            def _():
                pltpu.async_copy(xp0.at[srcb], xbuf, gsem).wait()

            @pl.when(c == 1)
            def _():
                pltpu.async_copy(xp1.at[srcb], xbuf, gsem).wait()

            for k in range(8):
                s16 = srcb[pl.ds(16 * k, 16)]
                d16 = dstb[pl.ds(16 * k, 16)]
                e = (plsc.load_gather(as_v, [s16])
                     + plsc.load_gather(ad_v, [d16]))
                e = jnp.where(e >= 0.0, e, 0.2 * e)
                sbuf[pl.ds(16 * k, 16)] = jnp.exp(e)
            pltpu.sync_copy(sbuf, den_sh.at[dstb], add=True)

            def _scale(r, _):
                a16 = plsc.load_gather(sbuf, [iota16 * 0 + r])
                for q in range(8):
                    xbuf[r, pl.ds(16 * q, 16)] = (
                        xbuf[r, pl.ds(16 * q, 16)] * a16)
                return 0

            lax.fori_loop(0, 128, _scale, 0)
            pltpu.sync_copy(xbuf, acc_sh.at[dstb], add=True)
            return 0

        lax.fori_loop(0, NCH, _chunk, 0)
        return 0

    lax.fori_loop(0, NIB, _iblk, 0)
    plsc.subcore_barrier()

    @pl.when(c == 0)
    def _():
        pltpu.sync_copy(den_sh.at[pl.ds(s * 640, 640)],
                        d0.at[pl.ds(s * 640, 640)])
        for jj in range(5):
            rows = pl.ds(s * 640 + jj * 128, 128)
            pltpu.sync_copy(acc_sh.at[rows], out0.at[rows])

    @pl.when(c == 1)
    def _():
        pltpu.sync_copy(den_sh.at[pl.ds(s * 640, 640)],
                        d1.at[pl.ds(s * 640, 640)])
        for jj in range(5):
            rows = pl.ds(s * 640 + jj * 128, 128)
            pltpu.sync_copy(acc_sh.at[rows], out1.at[rows])


def _sc_b(av, xp0, xp1, srch, dsth):
    mesh = plsc.VectorSubcoreMesh(core_axis_name="c", subcore_axis_name="s",
                                  num_cores=2, num_subcores=16)
    f = pl.kernel(
        _sc_b_body,
        out_type=[
            jax.ShapeDtypeStruct((NPAD, HIDDEN), jnp.float32),
            jax.ShapeDtypeStruct((NPAD, HIDDEN), jnp.float32),
            jax.ShapeDtypeStruct((NPAD,), jnp.float32),
            jax.ShapeDtypeStruct((NPAD,), jnp.float32),
        ],
        mesh=mesh,
        compiler_params=pltpu.CompilerParams(needs_layout_passes=False),
        scratch_types=[
            pltpu.VMEM((NPAD,), jnp.float32),
            pltpu.VMEM((NPAD,), jnp.float32),
            pltpu.VMEM((IBLK,), jnp.int32),
            pltpu.VMEM((IBLK,), jnp.int32),
            pltpu.VMEM((128,), jnp.int32),
            pltpu.VMEM((128,), jnp.int32),
            pltpu.VMEM((128,), jnp.float32),
            pltpu.VMEM((128, HIDDEN), jnp.float32),
            pltpu.VMEM((640,), jnp.float32),
            pltpu.VMEM_SHARED((NPAD,), jnp.float32),
            pltpu.VMEM_SHARED((NPAD, HIDDEN), jnp.float32),
            pltpu.SemaphoreType.DMA,
        ],
    )
    return f(av, xp0, xp1, srch, dsth)


def _tc_c_body(o0_ref, o1_ref, d0_ref, d1_ref, bias_ref, w1a_ref, w1b_ref,
               hb1_ref, u_ref, v_ref):
    r0 = 1.0 / (d0_ref[...] + 1e-16)
    r1 = 1.0 / (d1_ref[...] + 1e-16)
    h = (o0_ref[...] * r0 + o1_ref[...] * r1) * 0.5 + bias_ref[...]
    u_ref[...] = jnp.dot(h, w1a_ref[...],
                         preferred_element_type=jnp.float32) + hb1_ref[...]
    v_ref[...] = jnp.dot(h, w1b_ref[...],
                         preferred_element_type=jnp.float32) + hb1_ref[...]


def _tc_c(o0, o1, dn0, dn1, bias2d, W1a, W1b, hb1):
    return pl.pallas_call(
        _tc_c_body,
        grid=(NBLK,),
        in_specs=[
            pl.BlockSpec((NB, HIDDEN), lambda i: (i, 0)),
            pl.BlockSpec((NB, HIDDEN), lambda i: (i, 0)),
            pl.BlockSpec((NB, 1), lambda i: (i, 0)),
            pl.BlockSpec((NB, 1), lambda i: (i, 0)),
            pl.BlockSpec((1, HIDDEN), lambda i: (0, 0)),
            pl.BlockSpec((HIDDEN, 32), lambda i: (0, 0)),
            pl.BlockSpec((HIDDEN, 32), lambda i: (0, 0)),
            pl.BlockSpec((1, 32), lambda i: (0, 0)),
        ],
        out_specs=[
            pl.BlockSpec((NB, 32), lambda i: (i, 0)),
            pl.BlockSpec((NB, 32), lambda i: (i, 0)),
        ],
        out_shape=[
            jax.ShapeDtypeStruct((NPAD, 32), jnp.float32),
            jax.ShapeDtypeStruct((NPAD, 32), jnp.float32),
        ],
    )(o0, o1, dn0, dn1, bias2d, W1a, W1b, hb1)


def _sc_d_body(u_hbm, v_hbm, srcd, dstd, w2_hbm, b2_hbm, out,
               si_all, di_all, ub0, vb0, lb_all, w2v, b2v,
               u_sh, v_sh, gsem):
    c = lax.axis_index("c")
    s = lax.axis_index("s")
    wid = s * 2 + c
    base = wid * EP2_PER_W

    rows = pl.ds(s * 640, 640)
    pltpu.sync_copy(u_hbm.at[rows], u_sh.at[rows])
    pltpu.sync_copy(v_hbm.at[rows], v_sh.at[rows])

    pltpu.sync_copy(srcd.at[pl.ds(base, EP2_PER_W)], si_all)
    pltpu.sync_copy(dstd.at[pl.ds(base, EP2_PER_W)], di_all)
    pltpu.sync_copy(w2_hbm, w2v)
    pltpu.sync_copy(b2_hbm, b2v)
    plsc.subcore_barrier()

    iota16 = lax.iota(jnp.int32, 16)
    w2a = w2v[pl.ds(0, 16)]
    w2b = w2v[pl.ds(16, 16)]

    def _group(g, _):
        descs = []
        for q in range(4):
            off = g * 512 + q * 128
            descs.append(pltpu.async_copy(
                u_sh.at[si_all.at[pl.ds(off, 128)]],
                ub0.at[pl.ds(q * 128, 128)], gsem))
            descs.append(pltpu.async_copy(
                v_sh.at[di_all.at[pl.ds(off, 128)]],
                vb0.at[pl.ds(q * 128, 128)], gsem))
        for d in descs:
            d.wait()

        def _kk(kk, _):
            qe16 = (kk >> 3) * 128 + (kk & 7) * 16 + iota16
            accs = [None] * 4
            for f in range(32):
                f16 = jnp.full((16,), f, jnp.int32)
                z = (plsc.load_gather(ub0, [qe16, f16])
                     + plsc.load_gather(vb0, [qe16, f16]))
                z = jnp.maximum(z, 0.0)
                wf = w2a[f] if f < 16 else w2b[f - 16]
                j = f % 4
                accs[j] = z * wf if accs[j] is None else accs[j] + z * wf
            acc = (accs[0] + accs[1]) + (accs[2] + accs[3]) + b2v[...]
            lb_all[pl.ds(g * 512 + (kk >> 3) * 128 + (kk & 7) * 16, 16)] = acc
            return 0

        lax.fori_loop(0, 32, _kk, 0)
        return 0

    lax.fori_loop(0, EP2_GROUPS, _group, 0)
    pltpu.sync_copy(lb_all, out.at[pl.ds(base, EP2_PER_W)])


def _sc_d(u, v, srcd, dstd, w2p, b2p):
    mesh = plsc.VectorSubcoreMesh(core_axis_name="c", subcore_axis_name="s",
                                  num_cores=2, num_subcores=16)
    f = pl.kernel(
        _sc_d_body,
        out_type=jax.ShapeDtypeStruct((EP2,), jnp.float32),
        mesh=mesh,
        compiler_params=pltpu.CompilerParams(needs_layout_passes=False,
                                             use_tc_tiling_on_sc=False),
        scratch_types=[
            pltpu.VMEM((EP2_PER_W,), jnp.int32),
            pltpu.VMEM((EP2_PER_W,), jnp.int32),
            pltpu.VMEM((512, 32), jnp.float32),
            pltpu.VMEM((512, 32), jnp.float32),
            pltpu.VMEM((EP2_PER_W,), jnp.float32),
            pltpu.VMEM((32,), jnp.float32),
            pltpu.VMEM((16,), jnp.float32),
            pltpu.VMEM_SHARED((NPAD, 32), jnp.float32),
            pltpu.VMEM_SHARED((NPAD, 32), jnp.float32),
            pltpu.SemaphoreType.DMA,
        ],
    )
    return f(u, v, srcd, dstd, w2p, b2p)


def kernel(x, edge_index, W, att_src, att_dst, bias, W1, b1, W2, b2):
    src = edge_index[0]
    dst = edge_index[1]
    n_edges = src.shape[0]

    x_pad = jnp.pad(x, ((0, NPAD - N_NODES), (0, 0)))
    xT = x_pad.T

    Wr = W.reshape(IN_CH, HEADS, HIDDEN)
    pT = jnp.stack([
        Wr[:, 0, :] @ att_src[0],
        Wr[:, 1, :] @ att_src[1],
        Wr[:, 0, :] @ att_dst[0],
        Wr[:, 1, :] @ att_dst[1],
    ], axis=0)

    loop = jnp.arange(N_NODES, dtype=src.dtype)
    npad1 = EP1 - (n_edges + N_NODES)
    pad1 = N_NODES + (jnp.arange(npad1, dtype=src.dtype) % (NPAD - N_NODES))
    srch = jnp.concatenate([src, loop, pad1])
    dsth = jnp.concatenate([dst, loop, pad1])

    xp0, xp1, av = _tc_a(x_pad, W, xT, pT)
    o0, o1, dn0, dn1 = _sc_b(av, xp0, xp1, srch, dsth)

    u, v = _tc_c(o0, o1, dn0.reshape(NPAD, 1), dn1.reshape(NPAD, 1),
                 bias.reshape(1, HIDDEN), W1[:HIDDEN], W1[HIDDEN:],
                 (0.5 * b1).reshape(1, 32))

    npad2 = EP2 - n_edges
    pad2 = N_NODES + (jnp.arange(npad2, dtype=src.dtype) % (NPAD - N_NODES))
    srcd = jnp.concatenate([src, pad2])
    dstd = jnp.concatenate([dst, pad2])

    w2p = W2[:, 0] / TEMP
    b2p = jnp.full((16,), b2[0] / TEMP, jnp.float32)

    logits = _sc_d(u, v, srcd, dstd, w2p, b2p)
    return logits[:n_edges]


# speedup vs baseline: 1.2407x; 1.2407x over previous
import functools

import jax
import jax.numpy as jnp
from jax import lax
from jax.experimental import pallas as pl
from jax.experimental.pallas import tpu as pltpu
from jax.experimental.pallas import tpu_sc as plsc

N_NODES = 10000
NPAD = 10240
IN_CH = 128
HIDDEN = 128
HEADS = 2
TEMP = 0.7

EP1 = 331776
EP1_PER_TILE = EP1 // 16
EP1_CHUNKS = EP1_PER_TILE // 128

EP2 = 327680
EP2_PER_W = EP2 // 32
EP2_GROUPS = 20

NB = 512
NBLK = NPAD // NB


def _tc_a_body(x_ref, w_ref, xt_ref, pt_ref, xp0_ref, xp1_ref, av_ref):
    r = jnp.dot(x_ref[...], w_ref[...], preferred_element_type=jnp.float32)
    xp0_ref[...] = r[:, :HIDDEN]
    xp1_ref[...] = r[:, HIDDEN:]
    av_ref[...] = jnp.dot(pt_ref[...], xt_ref[...],
                          preferred_element_type=jnp.float32)


def _tc_a(x_pad, W, xT, pT):
    return pl.pallas_call(
        _tc_a_body,
        grid=(NBLK,),
        in_specs=[
            pl.BlockSpec((NB, IN_CH), lambda i: (i, 0)),
            pl.BlockSpec((IN_CH, HEADS * HIDDEN), lambda i: (0, 0)),
            pl.BlockSpec((IN_CH, NB), lambda i: (0, i)),
            pl.BlockSpec((4, IN_CH), lambda i: (0, 0)),
        ],
        out_specs=[
            pl.BlockSpec((NB, HIDDEN), lambda i: (i, 0)),
            pl.BlockSpec((NB, HIDDEN), lambda i: (i, 0)),
            pl.BlockSpec((4, NB), lambda i: (0, i)),
        ],
        out_shape=[
            jax.ShapeDtypeStruct((NPAD, HIDDEN), jnp.float32),
            jax.ShapeDtypeStruct((NPAD, HIDDEN), jnp.float32),
            jax.ShapeDtypeStruct((4, NPAD), jnp.float32),
        ],
    )(x_pad, W, xT, pT)


IBLK = 2304
SUBCH = EP1_PER_TILE // 64


def _sc_b_body(av, xp0, xp1, srch, dsth, out0, out1, d0, d1,
               as_v, ad_v, si2, di2, srcb0, dstb0, srcb1, dstb1,
               sb0, sb1, xb0, xb1, zrow,
               den_sh, acc_sh, gsem0, gsem1, ssem0, ssem1, dsem0, dsem1):
    c = lax.axis_index("c")
    s = lax.axis_index("s")
    zero16 = jnp.zeros((16,), jnp.float32)

    @pl.when(c == 0)
    def _():
        pltpu.sync_copy(av.at[0], as_v)
        pltpu.sync_copy(av.at[2], ad_v)

    @pl.when(c == 1)
    def _():
        pltpu.sync_copy(av.at[1], as_v)
        pltpu.sync_copy(av.at[3], ad_v)

    for i in range(40):
        zrow[pl.ds(16 * i, 16)] = zero16

    def _zrow_body(r, _):
        for q in range(8):
            xb0[r, pl.ds(16 * q, 16)] = zero16
            xb1[r, pl.ds(16 * q, 16)] = zero16
        return 0

    lax.fori_loop(0, 64, _zrow_body, 0)

    pltpu.sync_copy(zrow, den_sh.at[pl.ds(s * 640, 640)])
    for jj in range(10):
        pltpu.sync_copy(xb0 if jj % 2 == 0 else xb1,
                        acc_sh.at[pl.ds(s * 640 + jj * 64, 64)])
    plsc.subcore_barrier()

    ebase = s * EP1_PER_TILE
    iota16 = lax.iota(jnp.int32, 16)

    def _issue(g, srcb, dstb, xb, gsem, ssem, dsem):
        @pl.when(g >= 2)
        def _():
            pltpu.make_async_copy(xb, acc_sh.at[pl.ds(0, 64)], ssem).wait()
            pltpu.make_async_copy(srcb, den_sh.at[pl.ds(0, 64)], dsem).wait()
        off = (g % 36) * 64
        for k in range(4):
            srcb[pl.ds(16 * k, 16)] = si2[pl.ds(off + 16 * k, 16)]
            dstb[pl.ds(16 * k, 16)] = di2[pl.ds(off + 16 * k, 16)]

        @pl.when(c == 0)
        def _():
            pltpu.async_copy(xp0.at[srcb], xb, gsem)

        @pl.when(c == 1)
        def _():
            pltpu.async_copy(xp1.at[srcb], xb, gsem)

    def _work(srcb, dstb, xb, sb, gsem, ssem, dsem):
        pltpu.make_async_copy(xp0.at[pl.ds(0, 64)], xb, gsem).wait()
        for k in range(4):
            s16 = srcb[pl.ds(16 * k, 16)]
            d16 = dstb[pl.ds(16 * k, 16)]
            e = (plsc.load_gather(as_v, [s16])
                 + plsc.load_gather(ad_v, [d16]))
            e = jnp.where(e >= 0.0, e, 0.2 * e)
            sb[pl.ds(16 * k, 16)] = jnp.exp(e)
        pltpu.async_copy(sb, den_sh.at[dstb], dsem, add=True)

        def _scale(r, _):
            a16 = plsc.load_gather(sb, [iota16 * 0 + r])
            for q in range(8):
                xb[r, pl.ds(16 * q, 16)] = xb[r, pl.ds(16 * q, 16)] * a16
            return 0

        lax.fori_loop(0, 64, _scale, 0)
        pltpu.async_copy(xb, acc_sh.at[dstb], ssem, add=True)

    def _pipe(g, _):
        @pl.when((g % 36 == 0) & (g < SUBCH))
        def _():
            jo = g // 36
            pltpu.sync_copy(srch.at[pl.ds(ebase + jo * IBLK, IBLK)], si2)
            pltpu.sync_copy(dsth.at[pl.ds(ebase + jo * IBLK, IBLK)], di2)

        p = g % 2

        @pl.when(p == 0)
        def _():
            @pl.when(g < SUBCH)
            def _():
                _issue(g, srcb0, dstb0, xb0, gsem0, ssem0, dsem0)

            @pl.when(g > 0)
            def _():
                _work(srcb1, dstb1, xb1, sb1, gsem1, ssem1, dsem1)

        @pl.when(p == 1)
        def _():
            @pl.when(g < SUBCH)
            def _():
                _issue(g, srcb1, dstb1, xb1, gsem1, ssem1, dsem1)

            _work(srcb0, dstb0, xb0, sb0, gsem0, ssem0, dsem0)
        return 0

    lax.fori_loop(0, SUBCH + 1, _pipe, 0)
    pltpu.make_async_copy(xb0, acc_sh.at[pl.ds(0, 64)], ssem0).wait()
    pltpu.make_async_copy(xb1, acc_sh.at[pl.ds(0, 64)], ssem1).wait()
    pltpu.make_async_copy(srcb0, den_sh.at[pl.ds(0, 64)], dsem0).wait()
    pltpu.make_async_copy(srcb1, den_sh.at[pl.ds(0, 64)], dsem1).wait()
    plsc.subcore_barrier()

    @pl.when(c == 0)
    def _():
        pltpu.sync_copy(den_sh.at[pl.ds(s * 640, 640)],
                        d0.at[pl.ds(s * 640, 640)])
        for jj in range(5):
            rows = pl.ds(s * 640 + jj * 128, 128)
            pltpu.sync_copy(acc_sh.at[rows], out0.at[rows])

    @pl.when(c == 1)
    def _():
        pltpu.sync_copy(den_sh.at[pl.ds(s * 640, 640)],
                        d1.at[pl.ds(s * 640, 640)])
        for jj in range(5):
            rows = pl.ds(s * 640 + jj * 128, 128)
            pltpu.sync_copy(acc_sh.at[rows], out1.at[rows])


def _sc_b(av, xp0, xp1, srch, dsth):
    mesh = plsc.VectorSubcoreMesh(core_axis_name="c", subcore_axis_name="s",
                                  num_cores=2, num_subcores=16)
    f = pl.kernel(
        _sc_b_body,
        out_type=[
            jax.ShapeDtypeStruct((NPAD, HIDDEN), jnp.float32),
            jax.ShapeDtypeStruct((NPAD, HIDDEN), jnp.float32),
            jax.ShapeDtypeStruct((NPAD,), jnp.float32),
            jax.ShapeDtypeStruct((NPAD,), jnp.float32),
        ],
        mesh=mesh,
        compiler_params=pltpu.CompilerParams(needs_layout_passes=False),
        scratch_types=[
            pltpu.VMEM((NPAD,), jnp.float32),
            pltpu.VMEM((NPAD,), jnp.float32),
            pltpu.VMEM((IBLK,), jnp.int32),
            pltpu.VMEM((IBLK,), jnp.int32),
            pltpu.VMEM((64,), jnp.int32),
            pltpu.VMEM((64,), jnp.int32),
            pltpu.VMEM((64,), jnp.int32),
            pltpu.VMEM((64,), jnp.int32),
            pltpu.VMEM((64,), jnp.float32),
            pltpu.VMEM((64,), jnp.float32),
            pltpu.VMEM((64, HIDDEN), jnp.float32),
            pltpu.VMEM((64, HIDDEN), jnp.float32),
            pltpu.VMEM((640,), jnp.float32),
            pltpu.VMEM_SHARED((NPAD,), jnp.float32),
            pltpu.VMEM_SHARED((NPAD, HIDDEN), jnp.float32),
            pltpu.SemaphoreType.DMA,
            pltpu.SemaphoreType.DMA,
            pltpu.SemaphoreType.DMA,
            pltpu.SemaphoreType.DMA,
            pltpu.SemaphoreType.DMA,
            pltpu.SemaphoreType.DMA,
        ],
    )
    return f(av, xp0, xp1, srch, dsth)


def _tc_c_body(o0_ref, o1_ref, d0_ref, d1_ref, bias_ref, w1a_ref, w1b_ref,
               hb1_ref, u_ref, v_ref):
    r0 = 1.0 / (d0_ref[...] + 1e-16)
    r1 = 1.0 / (d1_ref[...] + 1e-16)
    h = (o0_ref[...] * r0 + o1_ref[...] * r1) * 0.5 + bias_ref[...]
    u_ref[...] = jnp.dot(h, w1a_ref[...],
                         preferred_element_type=jnp.float32) + hb1_ref[...]
    v_ref[...] = jnp.dot(h, w1b_ref[...],
                         preferred_element_type=jnp.float32) + hb1_ref[...]


def _tc_c(o0, o1, dn0, dn1, bias2d, W1a, W1b, hb1):
    return pl.pallas_call(
        _tc_c_body,
        grid=(NBLK,),
        in_specs=[
            pl.BlockSpec((NB, HIDDEN), lambda i: (i, 0)),
            pl.BlockSpec((NB, HIDDEN), lambda i: (i, 0)),
            pl.BlockSpec((NB, 1), lambda i: (i, 0)),
            pl.BlockSpec((NB, 1), lambda i: (i, 0)),
            pl.BlockSpec((1, HIDDEN), lambda i: (0, 0)),
            pl.BlockSpec((HIDDEN, 32), lambda i: (0, 0)),
            pl.BlockSpec((HIDDEN, 32), lambda i: (0, 0)),
            pl.BlockSpec((1, 32), lambda i: (0, 0)),
        ],
        out_specs=[
            pl.BlockSpec((NB, 32), lambda i: (i, 0)),
            pl.BlockSpec((NB, 32), lambda i: (i, 0)),
        ],
        out_shape=[
            jax.ShapeDtypeStruct((NPAD, 32), jnp.float32),
            jax.ShapeDtypeStruct((NPAD, 32), jnp.float32),
        ],
    )(o0, o1, dn0, dn1, bias2d, W1a, W1b, hb1)


def _sc_d_body(u_hbm, v_hbm, srcd, dstd, w2_hbm, b2_hbm, out,
               si_all, di_all, ub0, vb0, lb_all, w2v, b2v,
               u_sh, v_sh, gsem):
    c = lax.axis_index("c")
    s = lax.axis_index("s")
    wid = s * 2 + c
    base = wid * EP2_PER_W

    rows = pl.ds(s * 640, 640)
    pltpu.sync_copy(u_hbm.at[rows], u_sh.at[rows])
    pltpu.sync_copy(v_hbm.at[rows], v_sh.at[rows])

    pltpu.sync_copy(srcd.at[pl.ds(base, EP2_PER_W)], si_all)
    pltpu.sync_copy(dstd.at[pl.ds(base, EP2_PER_W)], di_all)
    pltpu.sync_copy(w2_hbm, w2v)
    pltpu.sync_copy(b2_hbm, b2v)
    plsc.subcore_barrier()

    iota16 = lax.iota(jnp.int32, 16)
    w2a = w2v[pl.ds(0, 16)]
    w2b = w2v[pl.ds(16, 16)]

    def _group(g, _):
        descs = []
        for q in range(4):
            off = g * 512 + q * 128
            descs.append(pltpu.async_copy(
                u_sh.at[si_all.at[pl.ds(off, 128)]],
                ub0.at[pl.ds(q * 128, 128)], gsem))
            descs.append(pltpu.async_copy(
                v_sh.at[di_all.at[pl.ds(off, 128)]],
                vb0.at[pl.ds(q * 128, 128)], gsem))
        for d in descs:
            d.wait()

        def _kk(kk, _):
            qe16 = (kk >> 3) * 128 + (kk & 7) * 16 + iota16
            accs = [None] * 4
            for f in range(32):
                f16 = jnp.full((16,), f, jnp.int32)
                z = (plsc.load_gather(ub0, [qe16, f16])
                     + plsc.load_gather(vb0, [qe16, f16]))
                z = jnp.maximum(z, 0.0)
                wf = w2a[f] if f < 16 else w2b[f - 16]
                j = f % 4
                accs[j] = z * wf if accs[j] is None else accs[j] + z * wf
            acc = (accs[0] + accs[1]) + (accs[2] + accs[3]) + b2v[...]
            lb_all[pl.ds(g * 512 + (kk >> 3) * 128 + (kk & 7) * 16, 16)] = acc
            return 0

        lax.fori_loop(0, 32, _kk, 0)
        return 0

    lax.fori_loop(0, EP2_GROUPS, _group, 0)
    pltpu.sync_copy(lb_all, out.at[pl.ds(base, EP2_PER_W)])


def _sc_d(u, v, srcd, dstd, w2p, b2p):
    mesh = plsc.VectorSubcoreMesh(core_axis_name="c", subcore_axis_name="s",
                                  num_cores=2, num_subcores=16)
    f = pl.kernel(
        _sc_d_body,
        out_type=jax.ShapeDtypeStruct((EP2,), jnp.float32),
        mesh=mesh,
        compiler_params=pltpu.CompilerParams(needs_layout_passes=False,
                                             use_tc_tiling_on_sc=False),
        scratch_types=[
            pltpu.VMEM((EP2_PER_W,), jnp.int32),
            pltpu.VMEM((EP2_PER_W,), jnp.int32),
            pltpu.VMEM((512, 32), jnp.float32),
            pltpu.VMEM((512, 32), jnp.float32),
            pltpu.VMEM((EP2_PER_W,), jnp.float32),
            pltpu.VMEM((32,), jnp.float32),
            pltpu.VMEM((16,), jnp.float32),
            pltpu.VMEM_SHARED((NPAD, 32), jnp.float32),
            pltpu.VMEM_SHARED((NPAD, 32), jnp.float32),
            pltpu.SemaphoreType.DMA,
        ],
    )
    return f(u, v, srcd, dstd, w2p, b2p)


def kernel(x, edge_index, W, att_src, att_dst, bias, W1, b1, W2, b2):
    src = edge_index[0]
    dst = edge_index[1]
    n_edges = src.shape[0]

    x_pad = jnp.pad(x, ((0, NPAD - N_NODES), (0, 0)))
    xT = x_pad.T

    Wr = W.reshape(IN_CH, HEADS, HIDDEN)
    pT = jnp.stack([
        Wr[:, 0, :] @ att_src[0],
        Wr[:, 1, :] @ att_src[1],
        Wr[:, 0, :] @ att_dst[0],
        Wr[:, 1, :] @ att_dst[1],
    ], axis=0)

    loop = jnp.arange(N_NODES, dtype=src.dtype)
    npad1 = EP1 - (n_edges + N_NODES)
    pad1 = N_NODES + (jnp.arange(npad1, dtype=src.dtype) % (NPAD - N_NODES))
    srch = jnp.concatenate([src, loop, pad1])
    dsth = jnp.concatenate([dst, loop, pad1])

    xp0, xp1, av = _tc_a(x_pad, W, xT, pT)
    o0, o1, dn0, dn1 = _sc_b(av, xp0, xp1, srch, dsth)

    u, v = _tc_c(o0, o1, dn0.reshape(NPAD, 1), dn1.reshape(NPAD, 1),
                 bias.reshape(1, HIDDEN), W1[:HIDDEN], W1[HIDDEN:],
                 (0.5 * b1).reshape(1, 32))

    npad2 = EP2 - n_edges
    pad2 = N_NODES + (jnp.arange(npad2, dtype=src.dtype) % (NPAD - N_NODES))
    srcd = jnp.concatenate([src, pad2])
    dstd = jnp.concatenate([dst, pad2])

    w2p = W2[:, 0] / TEMP
    b2p = jnp.full((16,), b2[0] / TEMP, jnp.float32)

    logits = _sc_d(u, v, srcd, dstd, w2p, b2p)
    return logits[:n_edges]


# speedup vs baseline: 1.3238x; 1.0670x over previous
import functools

import jax
import jax.numpy as jnp
from jax import lax
from jax.experimental import pallas as pl
from jax.experimental.pallas import tpu as pltpu
from jax.experimental.pallas import tpu_sc as plsc

N_NODES = 10000
NPAD = 10240
IN_CH = 128
HIDDEN = 128
HEADS = 2
TEMP = 0.7

EP1 = 331776
EP1_PER_TILE = EP1 // 16
EP1_CHUNKS = EP1_PER_TILE // 128

EP2 = 327680
EP2_PER_W = EP2 // 32
EP2_GROUPS = 20

NB = 512
NBLK = NPAD // NB


def _tc_a_body(x_ref, w_ref, xt_ref, pt_ref, xp0_ref, xp1_ref, av_ref):
    r = jnp.dot(x_ref[...], w_ref[...], preferred_element_type=jnp.float32)
    xp0_ref[...] = r[:, :HIDDEN]
    xp1_ref[...] = r[:, HIDDEN:]
    av_ref[...] = jnp.dot(pt_ref[...], xt_ref[...],
                          preferred_element_type=jnp.float32)


def _tc_a(x_pad, W, xT, pT):
    return pl.pallas_call(
        _tc_a_body,
        grid=(NBLK,),
        in_specs=[
            pl.BlockSpec((NB, IN_CH), lambda i: (i, 0)),
            pl.BlockSpec((IN_CH, HEADS * HIDDEN), lambda i: (0, 0)),
            pl.BlockSpec((IN_CH, NB), lambda i: (0, i)),
            pl.BlockSpec((4, IN_CH), lambda i: (0, 0)),
        ],
        out_specs=[
            pl.BlockSpec((NB, HIDDEN), lambda i: (i, 0)),
            pl.BlockSpec((NB, HIDDEN), lambda i: (i, 0)),
            pl.BlockSpec((4, NB), lambda i: (0, i)),
        ],
        out_shape=[
            jax.ShapeDtypeStruct((NPAD, HIDDEN), jnp.float32),
            jax.ShapeDtypeStruct((NPAD, HIDDEN), jnp.float32),
            jax.ShapeDtypeStruct((4, NPAD), jnp.float32),
        ],
    )(x_pad, W, xT, pT)


IBLK = 2304
SUBCH = EP1_PER_TILE // 64


def _sc_b_body(av, xp0, xp1, srch, dsth, out0, out1, d0, d1,
               as_v, ad_v, si2, di2, srcb0, dstb0, srcb1, dstb1,
               sb0, sb1, xb0, xb1, zrow,
               den_sh, acc_sh, gsem0, gsem1, ssem0, ssem1, dsem0, dsem1):
    c = lax.axis_index("c")
    s = lax.axis_index("s")
    zero16 = jnp.zeros((16,), jnp.float32)

    @pl.when(c == 0)
    def _():
        pltpu.sync_copy(av.at[0], as_v)
        pltpu.sync_copy(av.at[2], ad_v)

    @pl.when(c == 1)
    def _():
        pltpu.sync_copy(av.at[1], as_v)
        pltpu.sync_copy(av.at[3], ad_v)

    for i in range(40):
        zrow[pl.ds(16 * i, 16)] = zero16

    def _zrow_body(r, _):
        for q in range(8):
            xb0[r, pl.ds(16 * q, 16)] = zero16
            xb1[r, pl.ds(16 * q, 16)] = zero16
        return 0

    lax.fori_loop(0, 64, _zrow_body, 0)

    pltpu.sync_copy(zrow, den_sh.at[pl.ds(s * 640, 640)])
    for jj in range(10):
        pltpu.sync_copy(xb0 if jj % 2 == 0 else xb1,
                        acc_sh.at[pl.ds(s * 640 + jj * 64, 64)])
    plsc.subcore_barrier()

    ebase = s * EP1_PER_TILE
    iota16 = lax.iota(jnp.int32, 16)

    def _issue(g, srcb, dstb, xb, gsem, ssem, dsem):
        @pl.when(g >= 2)
        def _():
            pltpu.make_async_copy(xb, acc_sh.at[pl.ds(0, 64)], ssem).wait()
            pltpu.make_async_copy(srcb, den_sh.at[pl.ds(0, 64)], dsem).wait()
        off = (g % 36) * 64
        for k in range(4):
            srcb[pl.ds(16 * k, 16)] = si2[pl.ds(off + 16 * k, 16)]
            dstb[pl.ds(16 * k, 16)] = di2[pl.ds(off + 16 * k, 16)]

        @pl.when(c == 0)
        def _():
            pltpu.async_copy(xp0.at[srcb], xb, gsem)

        @pl.when(c == 1)
        def _():
            pltpu.async_copy(xp1.at[srcb], xb, gsem)

    def _work(srcb, dstb, xb, sb, gsem, ssem, dsem):
        pltpu.make_async_copy(xp0.at[pl.ds(0, 64)], xb, gsem).wait()
        for k in range(4):
            s16 = srcb[pl.ds(16 * k, 16)]
            d16 = dstb[pl.ds(16 * k, 16)]
            e = (plsc.load_gather(as_v, [s16])
                 + plsc.load_gather(ad_v, [d16]))
            e = jnp.where(e >= 0.0, e, 0.2 * e)
            sb[pl.ds(16 * k, 16)] = jnp.exp(e)
        pltpu.async_copy(sb, den_sh.at[dstb], dsem, add=True)

        def _scale(r, _):
            a16 = plsc.load_gather(sb, [iota16 * 0 + r])
            for q in range(8):
                xb[r, pl.ds(16 * q, 16)] = xb[r, pl.ds(16 * q, 16)] * a16
            return 0

        lax.fori_loop(0, 64, _scale, 0)
        pltpu.async_copy(xb, acc_sh.at[dstb], ssem, add=True)

    def _pipe(g, _):
        @pl.when((g % 36 == 0) & (g < SUBCH))
        def _():
            jo = g // 36
            pltpu.sync_copy(srch.at[pl.ds(ebase + jo * IBLK, IBLK)], si2)
            pltpu.sync_copy(dsth.at[pl.ds(ebase + jo * IBLK, IBLK)], di2)

        p = g % 2

        @pl.when(p == 0)
        def _():
            @pl.when(g < SUBCH)
            def _():
                _issue(g, srcb0, dstb0, xb0, gsem0, ssem0, dsem0)

            @pl.when(g > 0)
            def _():
                _work(srcb1, dstb1, xb1, sb1, gsem1, ssem1, dsem1)

        @pl.when(p == 1)
        def _():
            @pl.when(g < SUBCH)
            def _():
                _issue(g, srcb1, dstb1, xb1, gsem1, ssem1, dsem1)

            _work(srcb0, dstb0, xb0, sb0, gsem0, ssem0, dsem0)
        return 0

    lax.fori_loop(0, SUBCH + 1, _pipe, 0)
    pltpu.make_async_copy(xb0, acc_sh.at[pl.ds(0, 64)], ssem0).wait()
    pltpu.make_async_copy(xb1, acc_sh.at[pl.ds(0, 64)], ssem1).wait()
    pltpu.make_async_copy(srcb0, den_sh.at[pl.ds(0, 64)], dsem0).wait()
    pltpu.make_async_copy(srcb1, den_sh.at[pl.ds(0, 64)], dsem1).wait()
    plsc.subcore_barrier()

    @pl.when(c == 0)
    def _():
        pltpu.sync_copy(den_sh.at[pl.ds(s * 640, 640)],
                        d0.at[pl.ds(s * 640, 640)])
        for jj in range(5):
            rows = pl.ds(s * 640 + jj * 128, 128)
            pltpu.sync_copy(acc_sh.at[rows], out0.at[rows])

    @pl.when(c == 1)
    def _():
        pltpu.sync_copy(den_sh.at[pl.ds(s * 640, 640)],
                        d1.at[pl.ds(s * 640, 640)])
        for jj in range(5):
            rows = pl.ds(s * 640 + jj * 128, 128)
            pltpu.sync_copy(acc_sh.at[rows], out1.at[rows])


def _sc_b(av, xp0, xp1, srch, dsth):
    mesh = plsc.VectorSubcoreMesh(core_axis_name="c", subcore_axis_name="s",
                                  num_cores=2, num_subcores=16)
    f = pl.kernel(
        _sc_b_body,
        out_type=[
            jax.ShapeDtypeStruct((NPAD, HIDDEN), jnp.float32),
            jax.ShapeDtypeStruct((NPAD, HIDDEN), jnp.float32),
            jax.ShapeDtypeStruct((NPAD,), jnp.float32),
            jax.ShapeDtypeStruct((NPAD,), jnp.float32),
        ],
        mesh=mesh,
        compiler_params=pltpu.CompilerParams(needs_layout_passes=False),
        scratch_types=[
            pltpu.VMEM((NPAD,), jnp.float32),
            pltpu.VMEM((NPAD,), jnp.float32),
            pltpu.VMEM((IBLK,), jnp.int32),
            pltpu.VMEM((IBLK,), jnp.int32),
            pltpu.VMEM((64,), jnp.int32),
            pltpu.VMEM((64,), jnp.int32),
            pltpu.VMEM((64,), jnp.int32),
            pltpu.VMEM((64,), jnp.int32),
            pltpu.VMEM((64,), jnp.float32),
            pltpu.VMEM((64,), jnp.float32),
            pltpu.VMEM((64, HIDDEN), jnp.float32),
            pltpu.VMEM((64, HIDDEN), jnp.float32),
            pltpu.VMEM((640,), jnp.float32),
            pltpu.VMEM_SHARED((NPAD,), jnp.float32),
            pltpu.VMEM_SHARED((NPAD, HIDDEN), jnp.float32),
            pltpu.SemaphoreType.DMA,
            pltpu.SemaphoreType.DMA,
            pltpu.SemaphoreType.DMA,
            pltpu.SemaphoreType.DMA,
            pltpu.SemaphoreType.DMA,
            pltpu.SemaphoreType.DMA,
        ],
    )
    return f(av, xp0, xp1, srch, dsth)


def _tc_c_body(o0_ref, o1_ref, d0_ref, d1_ref, bias_ref, w1a_ref, w1b_ref,
               hb1_ref, u_ref, v_ref):
    r0 = 1.0 / (d0_ref[...] + 1e-16)
    r1 = 1.0 / (d1_ref[...] + 1e-16)
    h = (o0_ref[...] * r0 + o1_ref[...] * r1) * 0.5 + bias_ref[...]
    u_ref[...] = jnp.dot(h, w1a_ref[...],
                         preferred_element_type=jnp.float32) + hb1_ref[...]
    v_ref[...] = jnp.dot(h, w1b_ref[...],
                         preferred_element_type=jnp.float32) + hb1_ref[...]


def _tc_c(o0, o1, dn0, dn1, bias2d, W1a, W1b, hb1):
    return pl.pallas_call(
        _tc_c_body,
        grid=(NBLK,),
        in_specs=[
            pl.BlockSpec((NB, HIDDEN), lambda i: (i, 0)),
            pl.BlockSpec((NB, HIDDEN), lambda i: (i, 0)),
            pl.BlockSpec((NB, 1), lambda i: (i, 0)),
            pl.BlockSpec((NB, 1), lambda i: (i, 0)),
            pl.BlockSpec((1, HIDDEN), lambda i: (0, 0)),
            pl.BlockSpec((HIDDEN, 32), lambda i: (0, 0)),
            pl.BlockSpec((HIDDEN, 32), lambda i: (0, 0)),
            pl.BlockSpec((1, 32), lambda i: (0, 0)),
        ],
        out_specs=[
            pl.BlockSpec((NB, 32), lambda i: (i, 0)),
            pl.BlockSpec((NB, 32), lambda i: (i, 0)),
        ],
        out_shape=[
            jax.ShapeDtypeStruct((NPAD, 32), jnp.float32),
            jax.ShapeDtypeStruct((NPAD, 32), jnp.float32),
        ],
    )(o0, o1, dn0, dn1, bias2d, W1a, W1b, hb1)


def _sc_d_body(u_hbm, v_hbm, srcd, dstd, w2_hbm, b2_hbm, out,
               si_all, di_all, ub0, vb0, ub1, vb1, lb_all,
               w2v, b2v, sem0, sem1):
    c = lax.axis_index("c")
    s = lax.axis_index("s")
    wid = s * 2 + c
    base = wid * EP2_PER_W

    pltpu.sync_copy(srcd.at[pl.ds(base, EP2_PER_W)], si_all)
    pltpu.sync_copy(dstd.at[pl.ds(base, EP2_PER_W)], di_all)
    pltpu.sync_copy(w2_hbm, w2v)
    pltpu.sync_copy(b2_hbm, b2v)

    iota16 = lax.iota(jnp.int32, 16)
    w2a = w2v[pl.ds(0, 16)]
    w2b = w2v[pl.ds(16, 16)]

    def _issue(g, ub, vb, sem):
        for q in range(4):
            off = g * 512 + q * 128
            pltpu.async_copy(u_hbm.at[si_all.at[pl.ds(off, 128)]],
                             ub.at[pl.ds(q * 128, 128)], sem)
            pltpu.async_copy(v_hbm.at[di_all.at[pl.ds(off, 128)]],
                             vb.at[pl.ds(q * 128, 128)], sem)

    def _drain_compute(g, ub, vb, sem):
        pltpu.make_async_copy(u_hbm.at[pl.ds(0, 512)], ub, sem).wait()
        pltpu.make_async_copy(v_hbm.at[pl.ds(0, 512)], vb, sem).wait()

        def _kk(kk, _):
            qe16 = (kk >> 3) * 128 + (kk & 7) * 16 + iota16
            acc = b2v[...]
            for f in range(32):
                f16 = jnp.full((16,), f, jnp.int32)
                z = (plsc.load_gather(ub, [qe16, f16])
                     + plsc.load_gather(vb, [qe16, f16]))
                z = jnp.maximum(z, 0.0)
                wf = w2a[f] if f < 16 else w2b[f - 16]
                acc = acc + z * wf
            lb_all[pl.ds(g * 512 + (kk >> 3) * 128 + (kk & 7) * 16, 16)] = acc
            return 0

        lax.fori_loop(0, 32, _kk, 0)

    def _step(g, _):
        p = g % 2

        @pl.when(p == 0)
        def _():
            @pl.when(g < EP2_GROUPS)
            def _():
                _issue(g, ub0, vb0, sem0)

            @pl.when(g > 0)
            def _():
                _drain_compute(g - 1, ub1, vb1, sem1)

        @pl.when(p == 1)
        def _():
            @pl.when(g < EP2_GROUPS)
            def _():
                _issue(g, ub1, vb1, sem1)

            _drain_compute(g - 1, ub0, vb0, sem0)
        return 0

    lax.fori_loop(0, EP2_GROUPS + 1, _step, 0)
    pltpu.sync_copy(lb_all, out.at[pl.ds(base, EP2_PER_W)])


def _sc_d(u, v, srcd, dstd, w2p, b2p):
    mesh = plsc.VectorSubcoreMesh(core_axis_name="c", subcore_axis_name="s",
                                  num_cores=2, num_subcores=16)
    f = pl.kernel(
        _sc_d_body,
        out_type=jax.ShapeDtypeStruct((EP2,), jnp.float32),
        mesh=mesh,
        compiler_params=pltpu.CompilerParams(needs_layout_passes=False,
                                             use_tc_tiling_on_sc=False),
        scratch_types=[
            pltpu.VMEM((EP2_PER_W,), jnp.int32),
            pltpu.VMEM((EP2_PER_W,), jnp.int32),
            pltpu.VMEM((512, 32), jnp.float32),
            pltpu.VMEM((512, 32), jnp.float32),
            pltpu.VMEM((512, 32), jnp.float32),
            pltpu.VMEM((512, 32), jnp.float32),
            pltpu.VMEM((EP2_PER_W,), jnp.float32),
            pltpu.VMEM((32,), jnp.float32),
            pltpu.VMEM((16,), jnp.float32),
            pltpu.SemaphoreType.DMA,
            pltpu.SemaphoreType.DMA,
        ],
    )
    return f(u, v, srcd, dstd, w2p, b2p)


def kernel(x, edge_index, W, att_src, att_dst, bias, W1, b1, W2, b2):
    src = edge_index[0]
    dst = edge_index[1]
    n_edges = src.shape[0]

    x_pad = jnp.pad(x, ((0, NPAD - N_NODES), (0, 0)))
    xT = x_pad.T

    Wr = W.reshape(IN_CH, HEADS, HIDDEN)
    pT = jnp.stack([
        Wr[:, 0, :] @ att_src[0],
        Wr[:, 1, :] @ att_src[1],
        Wr[:, 0, :] @ att_dst[0],
        Wr[:, 1, :] @ att_dst[1],
    ], axis=0)

    loop = jnp.arange(N_NODES, dtype=src.dtype)
    npad1 = EP1 - (n_edges + N_NODES)
    pad1 = N_NODES + (jnp.arange(npad1, dtype=src.dtype) % (NPAD - N_NODES))
    srch = jnp.concatenate([src, loop, pad1])
    dsth = jnp.concatenate([dst, loop, pad1])

    xp0, xp1, av = _tc_a(x_pad, W, xT, pT)
    o0, o1, dn0, dn1 = _sc_b(av, xp0, xp1, srch, dsth)

    u, v = _tc_c(o0, o1, dn0.reshape(NPAD, 1), dn1.reshape(NPAD, 1),
                 bias.reshape(1, HIDDEN), W1[:HIDDEN], W1[HIDDEN:],
                 (0.5 * b1).reshape(1, 32))

    npad2 = EP2 - n_edges
    pad2 = N_NODES + (jnp.arange(npad2, dtype=src.dtype) % (NPAD - N_NODES))
    srcd = jnp.concatenate([src, pad2])
    dstd = jnp.concatenate([dst, pad2])

    w2p = W2[:, 0] / TEMP
    b2p = jnp.full((16,), b2[0] / TEMP, jnp.float32)

    logits = _sc_d(u, v, srcd, dstd, w2p, b2p)
    return logits[:n_edges]


# speedup vs baseline: 1.6992x; 1.2836x over previous
import functools

import jax
import jax.numpy as jnp
from jax import lax
from jax.experimental import pallas as pl
from jax.experimental.pallas import tpu as pltpu
from jax.experimental.pallas import tpu_sc as plsc

N_NODES = 10000
NPAD = 10240
IN_CH = 128
HIDDEN = 128
HEADS = 2
TEMP = 0.7

EP1 = 331776
EP1_PER_TILE = EP1 // 16
EP1_CHUNKS = EP1_PER_TILE // 128

EP2 = 327680
EP2_PER_W = EP2 // 32
EP2_GROUPS = 20

NB = 512
NBLK = NPAD // NB


def _tc_a_body(x_ref, w_ref, xt_ref, pt_ref, xp0_ref, xp1_ref, av_ref):
    r = jnp.dot(x_ref[...], w_ref[...], preferred_element_type=jnp.float32)
    xp0_ref[...] = r[:, :HIDDEN]
    xp1_ref[...] = r[:, HIDDEN:]
    av_ref[...] = jnp.dot(pt_ref[...], xt_ref[...],
                          preferred_element_type=jnp.float32)


def _tc_a(x_pad, W, xT, pT):
    return pl.pallas_call(
        _tc_a_body,
        grid=(NBLK,),
        in_specs=[
            pl.BlockSpec((NB, IN_CH), lambda i: (i, 0)),
            pl.BlockSpec((IN_CH, HEADS * HIDDEN), lambda i: (0, 0)),
            pl.BlockSpec((IN_CH, NB), lambda i: (0, i)),
            pl.BlockSpec((4, IN_CH), lambda i: (0, 0)),
        ],
        out_specs=[
            pl.BlockSpec((NB, HIDDEN), lambda i: (i, 0)),
            pl.BlockSpec((NB, HIDDEN), lambda i: (i, 0)),
            pl.BlockSpec((4, NB), lambda i: (0, i)),
        ],
        out_shape=[
            jax.ShapeDtypeStruct((NPAD, HIDDEN), jnp.float32),
            jax.ShapeDtypeStruct((NPAD, HIDDEN), jnp.float32),
            jax.ShapeDtypeStruct((4, NPAD), jnp.float32),
        ],
    )(x_pad, W, xT, pT)


IBLK = 2304
SUBCH = EP1_PER_TILE // 64


def _sc_b_body(av, xp0, xp1, srch, dsth, out0, out1, d0, d1,
               as_v, ad_v, si2, di2, srcb0, dstb0, srcb1, dstb1,
               sb0, sb1, xb0, xb1, zrow,
               den_sh, acc_sh, gsem0, gsem1, ssem0, ssem1, dsem0, dsem1):
    c = lax.axis_index("c")
    s = lax.axis_index("s")
    zero16 = jnp.zeros((16,), jnp.float32)

    @pl.when(c == 0)
    def _():
        pltpu.sync_copy(av.at[0], as_v)
        pltpu.sync_copy(av.at[2], ad_v)

    @pl.when(c == 1)
    def _():
        pltpu.sync_copy(av.at[1], as_v)
        pltpu.sync_copy(av.at[3], ad_v)

    for i in range(40):
        zrow[pl.ds(16 * i, 16)] = zero16

    def _zrow_body(r, _):
        for q in range(8):
            xb0[r, pl.ds(16 * q, 16)] = zero16
            xb1[r, pl.ds(16 * q, 16)] = zero16
        return 0

    lax.fori_loop(0, 64, _zrow_body, 0)

    pltpu.sync_copy(zrow, den_sh.at[pl.ds(s * 640, 640)])
    for jj in range(10):
        pltpu.sync_copy(xb0 if jj % 2 == 0 else xb1,
                        acc_sh.at[pl.ds(s * 640 + jj * 64, 64)])
    plsc.subcore_barrier()

    ebase = s * EP1_PER_TILE
    iota16 = lax.iota(jnp.int32, 16)

    def _issue(g, srcb, dstb, xb, gsem, ssem, dsem):
        @pl.when(g >= 2)
        def _():
            pltpu.make_async_copy(xb, acc_sh.at[pl.ds(0, 64)], ssem).wait()
            pltpu.make_async_copy(srcb, den_sh.at[pl.ds(0, 64)], dsem).wait()
        off = (g % 36) * 64
        for k in range(4):
            srcb[pl.ds(16 * k, 16)] = si2[pl.ds(off + 16 * k, 16)]
            dstb[pl.ds(16 * k, 16)] = di2[pl.ds(off + 16 * k, 16)]

        @pl.when(c == 0)
        def _():
            pltpu.async_copy(xp0.at[srcb], xb, gsem)

        @pl.when(c == 1)
        def _():
            pltpu.async_copy(xp1.at[srcb], xb, gsem)

    def _work(srcb, dstb, xb, sb, gsem, ssem, dsem):
        pltpu.make_async_copy(xp0.at[pl.ds(0, 64)], xb, gsem).wait()
        for k in range(4):
            s16 = srcb[pl.ds(16 * k, 16)]
            d16 = dstb[pl.ds(16 * k, 16)]
            e = (plsc.load_gather(as_v, [s16])
                 + plsc.load_gather(ad_v, [d16]))
            e = jnp.where(e >= 0.0, e, 0.2 * e)
            sb[pl.ds(16 * k, 16)] = jnp.exp(e)
        pltpu.async_copy(sb, den_sh.at[dstb], dsem, add=True)

        def _scale(r, _):
            a16 = plsc.load_gather(sb, [iota16 * 0 + r])
            for q in range(8):
                xb[r, pl.ds(16 * q, 16)] = xb[r, pl.ds(16 * q, 16)] * a16
            return 0

        lax.fori_loop(0, 64, _scale, 0)
        pltpu.async_copy(xb, acc_sh.at[dstb], ssem, add=True)

    def _pipe(g, _):
        @pl.when((g % 36 == 0) & (g < SUBCH))
        def _():
            jo = g // 36
            pltpu.sync_copy(srch.at[pl.ds(ebase + jo * IBLK, IBLK)], si2)
            pltpu.sync_copy(dsth.at[pl.ds(ebase + jo * IBLK, IBLK)], di2)

        p = g % 2

        @pl.when(p == 0)
        def _():
            @pl.when(g < SUBCH)
            def _():
                _issue(g, srcb0, dstb0, xb0, gsem0, ssem0, dsem0)

            @pl.when(g > 0)
            def _():
                _work(srcb1, dstb1, xb1, sb1, gsem1, ssem1, dsem1)

        @pl.when(p == 1)
        def _():
            @pl.when(g < SUBCH)
            def _():
                _issue(g, srcb1, dstb1, xb1, gsem1, ssem1, dsem1)

            _work(srcb0, dstb0, xb0, sb0, gsem0, ssem0, dsem0)
        return 0

    lax.fori_loop(0, SUBCH + 1, _pipe, 0)
    pltpu.make_async_copy(xb0, acc_sh.at[pl.ds(0, 64)], ssem0).wait()
    pltpu.make_async_copy(xb1, acc_sh.at[pl.ds(0, 64)], ssem1).wait()
    pltpu.make_async_copy(srcb0, den_sh.at[pl.ds(0, 64)], dsem0).wait()
    pltpu.make_async_copy(srcb1, den_sh.at[pl.ds(0, 64)], dsem1).wait()
    plsc.subcore_barrier()

    @pl.when(c == 0)
    def _():
        pltpu.sync_copy(den_sh.at[pl.ds(s * 640, 640)],
                        d0.at[pl.ds(s * 640, 640)])
        for jj in range(5):
            rows = pl.ds(s * 640 + jj * 128, 128)
            pltpu.sync_copy(acc_sh.at[rows], out0.at[rows])

    @pl.when(c == 1)
    def _():
        pltpu.sync_copy(den_sh.at[pl.ds(s * 640, 640)],
                        d1.at[pl.ds(s * 640, 640)])
        for jj in range(5):
            rows = pl.ds(s * 640 + jj * 128, 128)
            pltpu.sync_copy(acc_sh.at[rows], out1.at[rows])


def _sc_b(av, xp0, xp1, srch, dsth):
    mesh = plsc.VectorSubcoreMesh(core_axis_name="c", subcore_axis_name="s",
                                  num_cores=2, num_subcores=16)
    f = pl.kernel(
        _sc_b_body,
        out_type=[
            jax.ShapeDtypeStruct((NPAD, HIDDEN), jnp.float32),
            jax.ShapeDtypeStruct((NPAD, HIDDEN), jnp.float32),
            jax.ShapeDtypeStruct((NPAD,), jnp.float32),
            jax.ShapeDtypeStruct((NPAD,), jnp.float32),
        ],
        mesh=mesh,
        compiler_params=pltpu.CompilerParams(needs_layout_passes=False),
        scratch_types=[
            pltpu.VMEM((NPAD,), jnp.float32),
            pltpu.VMEM((NPAD,), jnp.float32),
            pltpu.VMEM((IBLK,), jnp.int32),
            pltpu.VMEM((IBLK,), jnp.int32),
            pltpu.VMEM((64,), jnp.int32),
            pltpu.VMEM((64,), jnp.int32),
            pltpu.VMEM((64,), jnp.int32),
            pltpu.VMEM((64,), jnp.int32),
            pltpu.VMEM((64,), jnp.float32),
            pltpu.VMEM((64,), jnp.float32),
            pltpu.VMEM((64, HIDDEN), jnp.float32),
            pltpu.VMEM((64, HIDDEN), jnp.float32),
            pltpu.VMEM((640,), jnp.float32),
            pltpu.VMEM_SHARED((NPAD,), jnp.float32),
            pltpu.VMEM_SHARED((NPAD, HIDDEN), jnp.float32),
            pltpu.SemaphoreType.DMA,
            pltpu.SemaphoreType.DMA,
            pltpu.SemaphoreType.DMA,
            pltpu.SemaphoreType.DMA,
            pltpu.SemaphoreType.DMA,
            pltpu.SemaphoreType.DMA,
        ],
    )
    return f(av, xp0, xp1, srch, dsth)


def _tc_c_body(o0_ref, o1_ref, d0_ref, d1_ref, bias_ref, w1at_ref, w1bt_ref,
               hb1_ref, ut_ref, vt_ref):
    r0 = 1.0 / (d0_ref[...] + 1e-16)
    r1 = 1.0 / (d1_ref[...] + 1e-16)
    h = (o0_ref[...] * r0 + o1_ref[...] * r1) * 0.5 + bias_ref[...]
    dn = (((1,), (1,)), ((), ()))
    ut_ref[...] = (lax.dot_general(w1at_ref[...], h, dn,
                                   preferred_element_type=jnp.float32)
                   + hb1_ref[...])
    vt_ref[...] = (lax.dot_general(w1bt_ref[...], h, dn,
                                   preferred_element_type=jnp.float32)
                   + hb1_ref[...])


def _tc_c(o0, o1, dn0, dn1, bias2d, W1aT, W1bT, hb1):
    return pl.pallas_call(
        _tc_c_body,
        grid=(NBLK,),
        in_specs=[
            pl.BlockSpec((NB, HIDDEN), lambda i: (i, 0)),
            pl.BlockSpec((NB, HIDDEN), lambda i: (i, 0)),
            pl.BlockSpec((NB, 1), lambda i: (i, 0)),
            pl.BlockSpec((NB, 1), lambda i: (i, 0)),
            pl.BlockSpec((1, HIDDEN), lambda i: (0, 0)),
            pl.BlockSpec((32, HIDDEN), lambda i: (0, 0)),
            pl.BlockSpec((32, HIDDEN), lambda i: (0, 0)),
            pl.BlockSpec((32, 1), lambda i: (0, 0)),
        ],
        out_specs=[
            pl.BlockSpec((32, NB), lambda i: (0, i)),
            pl.BlockSpec((32, NB), lambda i: (0, i)),
        ],
        out_shape=[
            jax.ShapeDtypeStruct((32, NPAD), jnp.float32),
            jax.ShapeDtypeStruct((32, NPAD), jnp.float32),
        ],
    )(o0, o1, dn0, dn1, bias2d, W1aT, W1bT, hb1)


EROWS = EP2 // 128
ESH = EP2 // 4
NBLK_D = ESH // 4096
IBLK_D = 4096


def _sc_d_body(ut_hbm, vt_hbm, srcd, dstd, w2_hbm, b2_hbm, out2d,
               ut_v, vt_v, sib, dib, lbuf, ridx, w2v, b2v, lg_sh, gsem):
    c = lax.axis_index("c")
    s = lax.axis_index("s")
    fg = s % 8
    es = (s // 8) * 2 + c
    ebase = es * ESH
    iota16 = lax.iota(jnp.int32, 16)
    zero16 = jnp.zeros((16,), jnp.float32)

    pltpu.sync_copy(ut_hbm.at[pl.ds(fg * 4, 4)], ut_v)
    pltpu.sync_copy(vt_hbm.at[pl.ds(fg * 4, 4)], vt_v)
    pltpu.sync_copy(w2_hbm, w2v)
    pltpu.sync_copy(b2_hbm, b2v)

    def _z(r, _):
        for q in range(8):
            lbuf[r, pl.ds(16 * q, 16)] = zero16
        return 0

    lax.fori_loop(0, 16, _z, 0)
    for t in range(10):
        pltpu.sync_copy(lbuf, lg_sh.at[pl.ds(s * 160 + t * 16, 16)])
    plsc.subcore_barrier()

    w2f = [plsc.load_gather(w2v, [iota16 * 0 + (fg * 4 + j)])
           for j in range(4)]
    binit = b2v[...] * jnp.where(fg == 0, 1.0, 0.0)

    def _iblk(b, _):
        pltpu.sync_copy(srcd.at[pl.ds(ebase + b * IBLK_D, IBLK_D)], sib)
        pltpu.sync_copy(dstd.at[pl.ds(ebase + b * IBLK_D, IBLK_D)], dib)

        def _blk(m, _):
            off = m * 2048
            def _kk(k, _):
                s16 = sib[pl.ds(off + 16 * k, 16)]
                d16 = dib[pl.ds(off + 16 * k, 16)]
                acc = binit
                for j in range(4):
                    j16 = jnp.full((16,), j, jnp.int32)
                    z = (plsc.load_gather(ut_v, [j16, s16])
                         + plsc.load_gather(vt_v, [j16, d16]))
                    acc = acc + jnp.maximum(z, 0.0) * w2f[j]
                lbuf[k >> 3, pl.ds((k & 7) * 16, 16)] = acc
                return 0

            lax.fori_loop(0, 128, _kk, 0)
            rbase = es * 640 + (b * 2 + m) * 16
            ridx[...] = iota16 + rbase
            pltpu.sync_copy(lbuf, lg_sh.at[ridx], add=True)
            return 0

        lax.fori_loop(0, 2, _blk, 0)
        return 0

    lax.fori_loop(0, NBLK_D, _iblk, 0)
    plsc.subcore_barrier()

    for half in range(2):
        rb = (c + 2 * half) * 640 + (s % 8) * 80 + (s // 8) * 40
        rows = pl.ds(rb, 40)
        pltpu.sync_copy(lg_sh.at[rows], out2d.at[rows])


def _sc_d(uT, vT, srcd, dstd, w2p, b2p):
    mesh = plsc.VectorSubcoreMesh(core_axis_name="c", subcore_axis_name="s",
                                  num_cores=2, num_subcores=16)
    f = pl.kernel(
        _sc_d_body,
        out_type=jax.ShapeDtypeStruct((EROWS, 128), jnp.float32),
        mesh=mesh,
        compiler_params=pltpu.CompilerParams(needs_layout_passes=False,
                                             use_tc_tiling_on_sc=False),
        scratch_types=[
            pltpu.VMEM((4, NPAD), jnp.float32),
            pltpu.VMEM((4, NPAD), jnp.float32),
            pltpu.VMEM((IBLK_D,), jnp.int32),
            pltpu.VMEM((IBLK_D,), jnp.int32),
            pltpu.VMEM((16, 128), jnp.float32),
            pltpu.VMEM((16,), jnp.int32),
            pltpu.VMEM((32,), jnp.float32),
            pltpu.VMEM((16,), jnp.float32),
            pltpu.VMEM_SHARED((EROWS, 128), jnp.float32),
            pltpu.SemaphoreType.DMA,
        ],
    )
    return f(uT, vT, srcd, dstd, w2p, b2p)


def kernel(x, edge_index, W, att_src, att_dst, bias, W1, b1, W2, b2):
    src = edge_index[0]
    dst = edge_index[1]
    n_edges = src.shape[0]

    x_pad = jnp.pad(x, ((0, NPAD - N_NODES), (0, 0)))
    xT = x_pad.T

    Wr = W.reshape(IN_CH, HEADS, HIDDEN)
    pT = jnp.stack([
        Wr[:, 0, :] @ att_src[0],
        Wr[:, 1, :] @ att_src[1],
        Wr[:, 0, :] @ att_dst[0],
        Wr[:, 1, :] @ att_dst[1],
    ], axis=0)

    loop = jnp.arange(N_NODES, dtype=src.dtype)
    npad1 = EP1 - (n_edges + N_NODES)
    pad1 = N_NODES + (jnp.arange(npad1, dtype=src.dtype) % (NPAD - N_NODES))
    srch = jnp.concatenate([src, loop, pad1])
    dsth = jnp.concatenate([dst, loop, pad1])

    xp0, xp1, av = _tc_a(x_pad, W, xT, pT)
    o0, o1, dn0, dn1 = _sc_b(av, xp0, xp1, srch, dsth)

    uT, vT = _tc_c(o0, o1, dn0.reshape(NPAD, 1), dn1.reshape(NPAD, 1),
                   bias.reshape(1, HIDDEN), W1[:HIDDEN].T, W1[HIDDEN:].T,
                   (0.5 * b1).reshape(32, 1))

    npad2 = EP2 - n_edges
    pad2 = N_NODES + (jnp.arange(npad2, dtype=src.dtype) % (NPAD - N_NODES))
    srcd = jnp.concatenate([src, pad2])
    dstd = jnp.concatenate([dst, pad2])

    w2p = W2[:, 0] / TEMP
    b2p = jnp.full((16,), b2[0] / TEMP, jnp.float32)

    logits2d = _sc_d(uT, vT, srcd, dstd, w2p, b2p)
    return logits2d.reshape(EP2)[:n_edges]
